# baseline (speedup 1.0000x reference)
import functools

import jax
import jax.numpy as jnp
from jax import lax
from jax.experimental import pallas as pl
from jax.experimental.pallas import tpu as pltpu
from jax.experimental.pallas import tpu_sc as plsc

N = 10000
E = 160000
D_EDGE = 16
EMBED = 64
SMP = 8
L = 16
NSUB = 16
C = 2 * E // NSUB
STRIPE = 624
TAIL = N - NSUB * STRIPE
KFIRE = 8
KROWS = (C + KFIRE * 128) // 128 + 2
APAD = N + 128

_NBLK = 1000
_EBLK = 8000

_SC_PARAMS = pltpu.CompilerParams(needs_layout_passes=False,
                                  use_tc_tiling_on_sc=False)


def _vtake(x, idx):
    dnums = lax.GatherDimensionNumbers(
        offset_dims=(), collapsed_slice_dims=(0,), start_index_map=(0,))
    return lax.gather(x, idx[:, None], dnums, slice_sizes=(1,),
                      mode=lax.GatherScatterMode.PROMISE_IN_BOUNDS)


def _dup_rank(nv, val, iota):
    key = jnp.bitwise_or(lax.shift_left(nv, 4), iota)
    s, sval = plsc.sort_key_val(key, val)
    sn = lax.shift_right_logical(s, 4)
    s_prev = _vtake(sn, jnp.maximum(iota - 1, 0))
    b = jnp.logical_or(iota == 0, sn != s_prev)
    start = plsc.cummax(jnp.where(b, iota, 0))
    fwd = iota - start + 1
    s_next = _vtake(sn, jnp.minimum(iota + 1, L - 1))
    last = jnp.logical_or(iota == L - 1, sn != s_next)
    return sn, sval, fwd, last


def _agg_body(nodes_hbm, x_hbm, accum_hbm, counts_hbm,
              nodes_v, run_v, tmp_v, ke_v, kn_v, feats_v,
              sem, sem2, hists_sh, accum_sh):
    wid = lax.axis_index("s")
    iota = lax.iota(jnp.int32, L)
    zero16i = jnp.zeros((L,), jnp.int32)

    pltpu.sync_copy(nodes_hbm.at[pl.ds(wid * C, C)], nodes_v)

    def _zrow(i, _):
        feats_v[i] = jnp.zeros((L,), jnp.float32)
        return 0
    lax.fori_loop(0, STRIPE, _zrow, 0)
    pltpu.sync_copy(feats_v.at[pl.ds(0, STRIPE)],
                    accum_sh.at[pl.ds(wid * STRIPE, STRIPE)])

    @pl.when(wid == 0)
    def _ztail():
        pltpu.sync_copy(feats_v.at[pl.ds(0, TAIL)],
                        accum_sh.at[pl.ds(NSUB * STRIPE, TAIL)])

    def _zero(i, _):
        run_v[pl.ds(i * L, L)] = zero16i
        return 0
    lax.fori_loop(0, N // L, _zero, 0)

    def _hist(i, _):
        nv = nodes_v[pl.ds(i * L, L)]
        sn, _, fwd, last = _dup_rank(nv, iota, iota)
        plsc.addupdate_scatter(run_v, [sn], fwd, mask=last)
        return 0
    lax.fori_loop(0, C // L, _hist, 0)

    for k in (1, 2, 4, 8):
        pltpu.sync_copy(run_v, hists_sh.at[wid])
        plsc.subcore_barrier()

        @pl.when(wid >= k)
        def _read():
            pltpu.sync_copy(hists_sh.at[wid - k], tmp_v)
        plsc.subcore_barrier()

        @pl.when(wid >= k)
        def _acc():
            def _add(i, _):
                sl = pl.ds(i * L, L)
                run_v[sl] = run_v[sl] + tmp_v[sl]
                return 0
            lax.fori_loop(0, N // L, _add, 0)

    @pl.when(wid == NSUB - 1)
    def _counts():
        pltpu.sync_copy(run_v, counts_hbm)

    pltpu.sync_copy(run_v, hists_sh.at[wid])
    plsc.subcore_barrier()

    @pl.when(wid > 0)
    def _base():
        pltpu.sync_copy(hists_sh.at[wid - 1], run_v)

    @pl.when(wid == 0)
    def _base0():
        def _z0(i, _):
            run_v[pl.ds(i * L, L)] = zero16i
            return 0
        lax.fori_loop(0, N // L, _z0, 0)

    def _rank(i, m):
        nv = nodes_v[pl.ds(i * L, L)]
        p = wid * C + i * L + iota
        e = jnp.where(p < E, p, p - E)
        sn, es, fwd, last = _dup_rank(nv, e, iota)
        old = plsc.load_gather(run_v, [sn])
        plsc.store_scatter(run_v, [sn], old + fwd, mask=last)
        rank = old + fwd - 1
        keep = rank < SMP
        cs = plsc.cumsum(keep.astype(jnp.int32))
        pos = m + cs - 1
        pr = lax.shift_right_logical(pos, 7)
        pc = jnp.bitwise_and(pos, 127)
        plsc.store_scatter(ke_v, [pr, pc], es, mask=keep)
        plsc.store_scatter(kn_v, [pr, pc], sn, mask=keep)
        return m + cs[L - 1]
    m = lax.fori_loop(0, C // L, _rank, jnp.int32(0))

    for t in range(KFIRE * 8):
        posp = m + t * L + iota
        pr = lax.shift_right_logical(posp, 7)
        pc = jnp.bitwise_and(posp, 127)
        plsc.store_scatter(ke_v, [pr, pc], zero16i)
        plsc.store_scatter(kn_v, [pr, pc], jnp.full((L,), N, jnp.int32))

    ngrp = (m + KFIRE * 128 - 1) // (KFIRE * 128)

    def _feed(g, _):
        descs = []
        for j in range(KFIRE):
            descs.append(pltpu.async_copy(
                x_hbm.at[ke_v.at[g * KFIRE + j]],
                feats_v.at[pl.ds(j * 128, 128)], sem))
        for d in descs:
            d.wait()
        descs = []
        for j in range(KFIRE):
            descs.append(pltpu.async_copy(
                feats_v.at[pl.ds(j * 128, 128)],
                accum_sh.at[kn_v.at[g * KFIRE + j]], sem2, add=True))
        for d in descs:
            d.wait()
        return 0
    lax.fori_loop(0, ngrp, _feed, 0)

    plsc.subcore_barrier()
    pltpu.sync_copy(accum_sh.at[pl.ds(wid * STRIPE, STRIPE)],
                    accum_hbm.at[pl.ds(wid * STRIPE, STRIPE)])

    @pl.when(wid == 0)
    def _otail():
        pltpu.sync_copy(accum_sh.at[pl.ds(NSUB * STRIPE, TAIL)],
                        accum_hbm.at[pl.ds(NSUB * STRIPE, TAIL)])


def _agg_call(nodes_flat, x):
    mesh = plsc.VectorSubcoreMesh(core_axis_name="c", subcore_axis_name="s",
                                  num_cores=1, num_subcores=NSUB)
    f = pl.kernel(
        _agg_body,
        out_type=[jax.ShapeDtypeStruct((N, D_EDGE), jnp.float32),
                  jax.ShapeDtypeStruct((N,), jnp.int32)],
        mesh=mesh,
        compiler_params=_SC_PARAMS,
        scratch_types=[
            pltpu.VMEM((C,), jnp.int32),
            pltpu.VMEM((N,), jnp.int32),
            pltpu.VMEM((N,), jnp.int32),
            pltpu.VMEM((KROWS, 128), jnp.int32),
            pltpu.VMEM((KROWS, 128), jnp.int32),
            pltpu.VMEM((KFIRE * 128, D_EDGE), jnp.float32),
            pltpu.SemaphoreType.DMA,
            pltpu.SemaphoreType.DMA,
            pltpu.VMEM_SHARED((NSUB, N), jnp.int32),
            pltpu.VMEM_SHARED((APAD, D_EDGE), jnp.float32),
        ],
    )
    return f(nodes_flat, x)


EW = 32
EPW = E // EW


def _edge_body(a_hbm, b_hbm, src_hbm, dst_hbm, q_hbm, out_hbm,
               a_v, b_v, src_v, dst_v, q_v, out_v):
    wid = lax.axis_index("s") * 2 + lax.axis_index("c")
    iota = lax.iota(jnp.int32, L)
    pltpu.sync_copy(a_hbm, a_v)
    pltpu.sync_copy(b_hbm, b_v)
    pltpu.sync_copy(src_hbm.at[pl.ds(wid * EPW, EPW)], src_v)
    pltpu.sync_copy(dst_hbm.at[pl.ds(wid * EPW, EPW)], dst_v)
    pltpu.sync_copy(q_hbm.at[pl.ds(wid * 2 * EPW, 2 * EPW)], q_v)

    half = lax.shift_right_logical(iota, 1)
    cls = jnp.bitwise_and(iota, 1)

    def _step(j, _):
        eidx = j * (L // 2) + half
        sv = plsc.load_gather(src_v, [eidx])
        dv = plsc.load_gather(dst_v, [eidx])
        ga = plsc.load_gather(a_v, [2 * sv + cls])
        gb = plsc.load_gather(b_v, [2 * dv + cls])
        sl = pl.ds(j * L, L)
        out_v[sl] = ga + gb + q_v[sl]
        return 0
    lax.fori_loop(0, 2 * EPW // L, _step, 0)
    pltpu.sync_copy(out_v, out_hbm.at[pl.ds(wid * 2 * EPW, 2 * EPW)])


def _edge_call(A, B, src, dst, q):
    mesh = plsc.VectorSubcoreMesh(core_axis_name="c", subcore_axis_name="s",
                                  num_cores=2, num_subcores=NSUB)
    f = pl.kernel(
        _edge_body,
        out_type=jax.ShapeDtypeStruct((2 * E,), jnp.float32),
        mesh=mesh,
        compiler_params=_SC_PARAMS,
        scratch_types=[
            pltpu.VMEM((2 * N,), jnp.float32),
            pltpu.VMEM((2 * N,), jnp.float32),
            pltpu.VMEM((EPW,), jnp.int32),
            pltpu.VMEM((EPW,), jnp.int32),
            pltpu.VMEM((2 * EPW,), jnp.float32),
            pltpu.VMEM((2 * EPW,), jnp.float32),
        ],
    )
    return f(A.reshape(2 * N), B.reshape(2 * N), src, dst, q.reshape(2 * E))


def _dense_body(accum_ref, cnt_ref, x_ref, w1_ref, w2_ref, wo_ref,
                a_ref, b_ref, q_ref):
    cnt = cnt_ref[...]
    k = jnp.minimum(cnt, float(SMP))
    neigh = jnp.where(cnt > 0.0, accum_ref[...] / jnp.maximum(k, 1.0), 0.01)
    w1 = w1_ref[...]
    b1 = jnp.sum(w1[:, :EMBED], axis=1)
    h1 = jax.nn.relu(
        jnp.dot(neigh, w1[:, EMBED:].T, preferred_element_type=jnp.float32)
        + b1[None, :])
    w2 = w2_ref[...]
    nodes = jax.nn.relu(
        jnp.dot(h1, w2[:, :EMBED].T, preferred_element_type=jnp.float32)
        + jnp.dot(neigh, w2[:, EMBED:].T, preferred_element_type=jnp.float32))
    wo = wo_ref[...]
    a_ref[...] = jnp.dot(nodes, wo[:, :EMBED].T,
                         preferred_element_type=jnp.float32)
    b_ref[...] = jnp.dot(nodes, wo[:, EMBED:2 * EMBED].T,
                         preferred_element_type=jnp.float32)
    q_ref[...] = jnp.dot(x_ref[...], wo[:, 2 * EMBED:].T,
                         preferred_element_type=jnp.float32)


def _dense_call(accum, counts_f, x, W1, W2, W_out):
    grid = N // _NBLK
    eblk = E // grid
    return pl.pallas_call(
        _dense_body,
        grid=(grid,),
        in_specs=[
            pl.BlockSpec((_NBLK, D_EDGE), lambda i: (i, 0)),
            pl.BlockSpec((_NBLK, 1), lambda i: (i, 0)),
            pl.BlockSpec((eblk, D_EDGE), lambda i: (i, 0)),
            pl.BlockSpec((EMBED, EMBED + D_EDGE), lambda i: (0, 0)),
            pl.BlockSpec((EMBED, EMBED + D_EDGE), lambda i: (0, 0)),
            pl.BlockSpec((2, 2 * EMBED + D_EDGE), lambda i: (0, 0)),
        ],
        out_specs=[
            pl.BlockSpec((_NBLK, 2), lambda i: (i, 0)),
            pl.BlockSpec((_NBLK, 2), lambda i: (i, 0)),
            pl.BlockSpec((eblk, 2), lambda i: (i, 0)),
        ],
        out_shape=[
            jax.ShapeDtypeStruct((N, 2), jnp.float32),
            jax.ShapeDtypeStruct((N, 2), jnp.float32),
            jax.ShapeDtypeStruct((E, 2), jnp.float32),
        ],
    )(accum, counts_f, x, W1, W2, W_out)


def kernel(x, adj_t, W1, W2, W_out):
    nodes_flat = adj_t.reshape(2 * E)
    accum, counts = _agg_call(nodes_flat, x)
    counts_f = counts.astype(jnp.float32)[:, None]
    A, B, q = _dense_call(accum, counts_f, x, W1, W2, W_out)
    scores = _edge_call(A, B, adj_t[0], adj_t[1], q)
    return scores.reshape(E, 2)


# speedup vs baseline: 1.3751x; 1.3751x over previous
import functools

import jax
import jax.numpy as jnp
from jax import lax
from jax.experimental import pallas as pl
from jax.experimental.pallas import tpu as pltpu
from jax.experimental.pallas import tpu_sc as plsc

N = 10000
E = 160000
D_EDGE = 16
EMBED = 64
SMP = 8
L = 16
NSUB = 16
C = 2 * E // NSUB
STRIPE = 624
TAIL = N - NSUB * STRIPE
KFIRE = 8
KROWS = (C + KFIRE * 128) // 128 + 2
APAD = N + 128

_NBLK = 1000
_EBLK = 8000
_PBLK = 6400

_SC_PARAMS = pltpu.CompilerParams(needs_layout_passes=False,
                                  use_tc_tiling_on_sc=False)


def _vtake(x, idx):
    dnums = lax.GatherDimensionNumbers(
        offset_dims=(), collapsed_slice_dims=(0,), start_index_map=(0,))
    return lax.gather(x, idx[:, None], dnums, slice_sizes=(1,),
                      mode=lax.GatherScatterMode.PROMISE_IN_BOUNDS)


def _dup_rank(nv, val, iota):
    key = jnp.bitwise_or(lax.shift_left(nv, 4), iota)
    s, sval = plsc.sort_key_val(key, val)
    sn = lax.shift_right_logical(s, 4)
    s_prev = _vtake(sn, jnp.maximum(iota - 1, 0))
    b = jnp.logical_or(iota == 0, sn != s_prev)
    start = plsc.cummax(jnp.where(b, iota, 0))
    fwd = iota - start + 1
    s_next = _vtake(sn, jnp.minimum(iota + 1, L - 1))
    last = jnp.logical_or(iota == L - 1, sn != s_next)
    return sn, sval, fwd, last


def _agg_body(adj_hbm, x_hbm, accum_hbm, counts_hbm,
              nodes_v, run_v, tmp_v, ke_v, kn_v, feats_v,
              sem, sem2, hists_sh, accum_sh):
    wid = lax.axis_index("s")
    iota = lax.iota(jnp.int32, L)
    zero16i = jnp.zeros((L,), jnp.int32)

    row = lax.shift_right_logical(wid, 3)
    col0 = jnp.bitwise_and(wid, 7) * C
    pltpu.sync_copy(adj_hbm.at[row, pl.ds(col0, C)], nodes_v)

    def _zrow(i, _):
        feats_v[i] = jnp.zeros((L,), jnp.float32)
        return 0
    lax.fori_loop(0, STRIPE, _zrow, 0)
    pltpu.sync_copy(feats_v.at[pl.ds(0, STRIPE)],
                    accum_sh.at[pl.ds(wid * STRIPE, STRIPE)])

    @pl.when(wid == 0)
    def _ztail():
        pltpu.sync_copy(feats_v.at[pl.ds(0, TAIL)],
                        accum_sh.at[pl.ds(NSUB * STRIPE, TAIL)])

    def _zero(i, _):
        run_v[pl.ds(i * L, L)] = zero16i
        return 0
    lax.fori_loop(0, N // L, _zero, 0)

    def _hist(i, _):
        nv = nodes_v[pl.ds(i * L, L)]
        sn, _, fwd, last = _dup_rank(nv, iota, iota)
        plsc.addupdate_scatter(run_v, [sn], fwd, mask=last)
        return 0
    lax.fori_loop(0, C // L, _hist, 0)

    for k in (1, 2, 4, 8):
        pltpu.sync_copy(run_v, hists_sh.at[wid])
        plsc.subcore_barrier()

        @pl.when(wid >= k)
        def _read():
            pltpu.sync_copy(hists_sh.at[wid - k], tmp_v)
        plsc.subcore_barrier()

        @pl.when(wid >= k)
        def _acc():
            def _add(i, _):
                sl = pl.ds(i * L, L)
                run_v[sl] = run_v[sl] + tmp_v[sl]
                return 0
            lax.fori_loop(0, N // L, _add, 0)

    @pl.when(wid == NSUB - 1)
    def _counts():
        pltpu.sync_copy(run_v, counts_hbm)

    pltpu.sync_copy(run_v, hists_sh.at[wid])
    plsc.subcore_barrier()

    @pl.when(wid > 0)
    def _base():
        pltpu.sync_copy(hists_sh.at[wid - 1], run_v)

    @pl.when(wid == 0)
    def _base0():
        def _z0(i, _):
            run_v[pl.ds(i * L, L)] = zero16i
            return 0
        lax.fori_loop(0, N // L, _z0, 0)

    def _rank(i, m):
        nv = nodes_v[pl.ds(i * L, L)]
        p = wid * C + i * L + iota
        e = jnp.where(p < E, p, p - E)
        sn, es, fwd, last = _dup_rank(nv, e, iota)
        old = plsc.load_gather(run_v, [sn])
        plsc.store_scatter(run_v, [sn], old + fwd, mask=last)
        rank = old + fwd - 1
        keep = rank < SMP
        cs = plsc.cumsum(keep.astype(jnp.int32))
        pos = m + cs - 1
        pr = lax.shift_right_logical(pos, 7)
        pc = jnp.bitwise_and(pos, 127)
        plsc.store_scatter(ke_v, [pr, pc], es, mask=keep)
        plsc.store_scatter(kn_v, [pr, pc], sn, mask=keep)
        return m + cs[L - 1]
    m = lax.fori_loop(0, C // L, _rank, jnp.int32(0))

    for t in range(KFIRE * 8):
        posp = m + t * L + iota
        pr = lax.shift_right_logical(posp, 7)
        pc = jnp.bitwise_and(posp, 127)
        plsc.store_scatter(ke_v, [pr, pc], zero16i)
        plsc.store_scatter(kn_v, [pr, pc], jnp.full((L,), N, jnp.int32))

    ngrp = (m + KFIRE * 128 - 1) // (KFIRE * 128)

    def _feed(g, _):
        descs = []
        for j in range(KFIRE):
            descs.append(pltpu.async_copy(
                x_hbm.at[ke_v.at[g * KFIRE + j]],
                feats_v.at[pl.ds(j * 128, 128)], sem))
        for d in descs:
            d.wait()
        descs = []
        for j in range(KFIRE):
            descs.append(pltpu.async_copy(
                feats_v.at[pl.ds(j * 128, 128)],
                accum_sh.at[kn_v.at[g * KFIRE + j]], sem2, add=True))
        for d in descs:
            d.wait()
        return 0
    lax.fori_loop(0, ngrp, _feed, 0)

    plsc.subcore_barrier()
    pltpu.sync_copy(accum_sh.at[pl.ds(wid * STRIPE, STRIPE)],
                    accum_hbm.at[pl.ds(wid * STRIPE, STRIPE)])

    @pl.when(wid == 0)
    def _otail():
        pltpu.sync_copy(accum_sh.at[pl.ds(NSUB * STRIPE, TAIL)],
                        accum_hbm.at[pl.ds(NSUB * STRIPE, TAIL)])


def _agg_call(adj_t, x):
    mesh = plsc.VectorSubcoreMesh(core_axis_name="c", subcore_axis_name="s",
                                  num_cores=1, num_subcores=NSUB)
    f = pl.kernel(
        _agg_body,
        out_type=[jax.ShapeDtypeStruct((N, D_EDGE), jnp.float32),
                  jax.ShapeDtypeStruct((N,), jnp.int32)],
        mesh=mesh,
        compiler_params=_SC_PARAMS,
        scratch_types=[
            pltpu.VMEM((C,), jnp.int32),
            pltpu.VMEM((N,), jnp.int32),
            pltpu.VMEM((N,), jnp.int32),
            pltpu.VMEM((KROWS, 128), jnp.int32),
            pltpu.VMEM((KROWS, 128), jnp.int32),
            pltpu.VMEM((KFIRE * 128, D_EDGE), jnp.float32),
            pltpu.SemaphoreType.DMA,
            pltpu.SemaphoreType.DMA,
            pltpu.VMEM_SHARED((NSUB, N), jnp.int32),
            pltpu.VMEM_SHARED((APAD, D_EDGE), jnp.float32),
        ],
    )
    return f(adj_t, x)


EW = 32
EPW = E // EW
EPWP = EPW + L
EHALF_A = 2496
EHALF_B = EPW - EHALF_A
XROWS = 2512
ASTEPS = EHALF_A // L
BSTEPS = (EHALF_B + L - 1) // L


def _edge_body(a0_hbm, a1_hbm, b0_hbm, b1_hbm, adj_hbm, x_hbm, wo_hbm,
               s0_hbm, s1_hbm,
               a0_v, a1_v, b0_v, b1_v, src_v, dst_v, xb_v, w_v,
               s0_v, s1_v):
    wid = lax.axis_index("s") * 2 + lax.axis_index("c")
    iota = lax.iota(jnp.int32, L)
    pltpu.sync_copy(a0_hbm, a0_v)
    pltpu.sync_copy(a1_hbm, a1_v)
    pltpu.sync_copy(b0_hbm, b0_v)
    pltpu.sync_copy(b1_hbm, b1_v)
    pltpu.sync_copy(wo_hbm.at[0, pl.ds(2 * EMBED, D_EDGE)], w_v.at[pl.ds(0, L)])
    pltpu.sync_copy(wo_hbm.at[1, pl.ds(2 * EMBED, D_EDGE)], w_v.at[pl.ds(L, L)])
    base = wid * EPW
    pltpu.sync_copy(adj_hbm.at[0, pl.ds(base, EPW)], src_v.at[pl.ds(0, EPW)])
    pltpu.sync_copy(adj_hbm.at[1, pl.ds(base, EPW)], dst_v.at[pl.ds(0, EPW)])
    zpad = jnp.zeros((L,), jnp.int32)
    plsc.store_scatter(src_v, [EPW + iota], zpad)
    plsc.store_scatter(dst_v, [EPW + iota], zpad)
    w0 = w_v[pl.ds(0, L)]
    w1 = w_v[pl.ds(L, L)]

    def _step(j, lb):
        sl = pl.ds(j * L, L)
        sv = src_v[sl]
        dv = dst_v[sl]
        rows = lb + iota
        q0 = jnp.zeros((L,), jnp.float32)
        q1 = jnp.zeros((L,), jnp.float32)
        for c in range(D_EDGE):
            xc = plsc.load_gather(xb_v, [rows, jnp.full((L,), c, jnp.int32)])
            q0 = q0 + xc * w0[c]
            q1 = q1 + xc * w1[c]
        s0_v[sl] = (plsc.load_gather(a0_v, [sv])
                    + plsc.load_gather(b0_v, [dv]) + q0)
        s1_v[sl] = (plsc.load_gather(a1_v, [sv])
                    + plsc.load_gather(b1_v, [dv]) + q1)

    pltpu.sync_copy(x_hbm.at[pl.ds(base, EHALF_A)],
                    xb_v.at[pl.ds(0, EHALF_A)])

    def _loop_a(j, _):
        _step(j, j * L)
        return 0
    lax.fori_loop(0, ASTEPS, _loop_a, 0)

    pltpu.sync_copy(x_hbm.at[pl.ds(base + EHALF_A, EHALF_B)],
                    xb_v.at[pl.ds(0, EHALF_B)])

    def _loop_b(j, _):
        _step(ASTEPS + j, j * L)
        return 0
    lax.fori_loop(0, BSTEPS, _loop_b, 0)

    pltpu.sync_copy(s0_v.at[pl.ds(0, EPW)], s0_hbm.at[pl.ds(base, EPW)])
    pltpu.sync_copy(s1_v.at[pl.ds(0, EPW)], s1_hbm.at[pl.ds(base, EPW)])


def _edge_call(a0, a1, b0, b1, adj_t, x, W_out):
    mesh = plsc.VectorSubcoreMesh(core_axis_name="c", subcore_axis_name="s",
                                  num_cores=2, num_subcores=NSUB)
    f = pl.kernel(
        _edge_body,
        out_type=[jax.ShapeDtypeStruct((E,), jnp.float32),
                  jax.ShapeDtypeStruct((E,), jnp.float32)],
        mesh=mesh,
        compiler_params=_SC_PARAMS,
        scratch_types=[
            pltpu.VMEM((N,), jnp.float32),
            pltpu.VMEM((N,), jnp.float32),
            pltpu.VMEM((N,), jnp.float32),
            pltpu.VMEM((N,), jnp.float32),
            pltpu.VMEM((EPWP,), jnp.int32),
            pltpu.VMEM((EPWP,), jnp.int32),
            pltpu.VMEM((XROWS, D_EDGE), jnp.float32),
            pltpu.VMEM((2 * L,), jnp.float32),
            pltpu.VMEM((EPWP,), jnp.float32),
            pltpu.VMEM((EPWP,), jnp.float32),
        ],
    )
    return f(a0, a1, b0, b1, adj_t, x, W_out)


def _dense_body(accum_ref, cnt_ref, w1_ref, w2_ref, wo_ref,
                a0_ref, a1_ref, b0_ref, b1_ref):
    cnt = cnt_ref[...]
    k = jnp.minimum(cnt, float(SMP))
    neigh = jnp.where(cnt > 0.0, accum_ref[...] / jnp.maximum(k, 1.0), 0.01)
    w1 = w1_ref[...]
    b1 = jnp.sum(w1[:, :EMBED], axis=1)
    h1 = jax.nn.relu(
        jnp.dot(neigh, w1[:, EMBED:].T, preferred_element_type=jnp.float32)
        + b1[None, :])
    w2 = w2_ref[...]
    nodes = jax.nn.relu(
        jnp.dot(h1, w2[:, :EMBED].T, preferred_element_type=jnp.float32)
        + jnp.dot(neigh, w2[:, EMBED:].T, preferred_element_type=jnp.float32))
    wo = wo_ref[...]
    a = jnp.dot(nodes, wo[:, :EMBED].T, preferred_element_type=jnp.float32)
    b = jnp.dot(nodes, wo[:, EMBED:2 * EMBED].T,
                preferred_element_type=jnp.float32)
    a0_ref[...] = a[:, 0]
    a1_ref[...] = a[:, 1]
    b0_ref[...] = b[:, 0]
    b1_ref[...] = b[:, 1]


def _dense_call(accum, counts_f, W1, W2, W_out):
    return pl.pallas_call(
        _dense_body,
        out_shape=[
            jax.ShapeDtypeStruct((N,), jnp.float32),
            jax.ShapeDtypeStruct((N,), jnp.float32),
            jax.ShapeDtypeStruct((N,), jnp.float32),
            jax.ShapeDtypeStruct((N,), jnp.float32),
        ],
    )(accum, counts_f, W1, W2, W_out)


def _pack_body(s0_ref, s1_ref, out_ref):
    i = pl.program_id(0)
    sl = pl.ds(i * _PBLK, _PBLK)
    out_ref[:, 0] = s0_ref[sl]
    out_ref[:, 1] = s1_ref[sl]


def _pack_call(s0, s1):
    grid = E // _PBLK
    return pl.pallas_call(
        _pack_body,
        grid=(grid,),
        in_specs=[
            pl.BlockSpec((E,), lambda i: (0,)),
            pl.BlockSpec((E,), lambda i: (0,)),
        ],
        out_specs=pl.BlockSpec((_PBLK, 2), lambda i: (i, 0)),
        out_shape=jax.ShapeDtypeStruct((E, 2), jnp.float32),
    )(s0, s1)


def kernel(x, adj_t, W1, W2, W_out):
    accum, counts = _agg_call(adj_t, x)
    counts_f = counts.astype(jnp.float32)[:, None]
    a0, a1, b0, b1 = _dense_call(accum, counts_f, W1, W2, W_out)
    s0, s1 = _edge_call(a0, a1, b0, b1, adj_t, x, W_out)
    return _pack_call(s0, s1)
